# baseline (speedup 1.0000x reference)
import jax
import jax.numpy as jnp
from jax import lax
from jax.experimental import pallas as pl
from jax.experimental.pallas import tpu as pltpu
from jax.experimental.pallas import tpu_sc as plsc

NC = 2
NS = 16
NW = NC * NS

EMBED_DIM = 32
BATCH = 16384
FIELDS = 26
KROWS = 256
NBLK = BATCH // KROWS
BLK_PER_W = NBLK // NW
SUB = 128
NSUB = KROWS // SUB
NGRP = KROWS // 16


def _gather_body(idx_hbm, table_hbm, out_hbm,
                 idxv0, idxv1, srv0, srv1, sst0, sst1, tb0, tb1,
                 gsem0, gsem1, osem0, osem1):
    wid = lax.axis_index("s") * NC + lax.axis_index("c")
    idxv = (idxv0, idxv1)
    srv = (srv0, srv1)
    sst = (sst0, sst1)
    tbuf = (tb0, tb1)
    gsem = (gsem0, gsem1)
    osem = (osem0, osem1)
    b0s = (wid * KROWS, (wid + NW) * KROWS)

    lane = lax.broadcasted_iota(jnp.int32, (16,), 0)

    def load_idx(f, p):
        pltpu.sync_copy(idx_hbm.at[f, pl.ds(b0s[p], KROWS)], idxv[p])

    def compute_srows(p):
        @plsc.parallel_loop(0, NGRP, unroll=4)
        def _(g):
            v = idxv[p][pl.ds(g * 16, 16)]
            srv[p][pl.ds(g * 16, 16)] = lax.shift_right_logical(v, 2)

    def fire_gather(p):
        for j in range(NSUB):
            pltpu.async_copy(
                table_hbm.at[srv[p].at[pl.ds(j * SUB, SUB)]],
                sst[p].at[pl.ds(j * SUB, SUB)],
                gsem[p])

    def drain_gather(p):
        pltpu.make_async_copy(
            table_hbm.at[srv[p]], sst[p], gsem[p]).wait()

    def extract(p):
        @plsc.parallel_loop(0, NGRP, unroll=2)
        def _(g):
            iv = idxv[p][pl.ds(g * 16, 16)]
            row = g * 16 + lane
            qcol = lax.shift_left(lax.bitwise_and(iv, 3), 5)
            for e in range(EMBED_DIM):
                vals = plsc.load_gather(sst[p], [row, qcol + e])
                tbuf[p][e, pl.ds(g * 16, 16)] = vals

    def fire_out(f, p):
        pltpu.async_copy(
            tbuf[p], out_hbm.at[f, :, pl.ds(b0s[p], KROWS)], osem[p])

    def drain_out(p):
        pltpu.make_async_copy(
            tbuf[p], out_hbm.at[0, :, pl.ds(b0s[p], KROWS)], osem[p]).wait()

    for p in (0, 1):
        load_idx(0, p)
        compute_srows(p)
        fire_gather(p)

    def field_body(f, carry):
        for p in (0, 1):
            drain_gather(p)
            pl.when(f >= 1)(lambda: drain_out(p))
            extract(p)
            fire_out(f, p)

            def prefetch(p=p):
                load_idx(f + 1, p)
                compute_srows(p)
                fire_gather(p)
            pl.when(f + 1 < FIELDS)(prefetch)
        return carry

    lax.fori_loop(0, FIELDS, field_body, 0)
    drain_out(0)
    drain_out(1)


NSR = 250000
TSR = 128
NT_FULL = NSR // TSR
TAIL = NSR - NT_FULL * TSR


def _transpose_body(tab_t, out_sr, src0, src1, ob0, ob1, tsrc,
                    isem0, isem1, osem0, osem1):
    wid = lax.axis_index("s") * NC + lax.axis_index("c")
    src = (src0, src1)
    obuf = (ob0, ob1)
    isem = (isem0, isem1)
    osem = (osem0, osem1)
    lane = lax.broadcasted_iota(jnp.int32, (16,), 0)

    def fire_in(t, p):
        for e in range(EMBED_DIM):
            pltpu.async_copy(
                tab_t.at[e, pl.ds(t * (4 * TSR), 4 * TSR)],
                src[p].at[pl.ds(e * (4 * TSR), 4 * TSR)], isem[p])

    def drain_in(p):
        pltpu.make_async_copy(
            tab_t.at[0, pl.ds(0, EMBED_DIM * 4 * TSR)], src[p],
            isem[p]).wait()

    def transpose_ref(sref, oref, nsr, stride):
        elane0 = lane * stride
        elane16 = lane * stride + 16 * stride
        @plsc.parallel_loop(0, nsr, unroll=8)
        def _(s):
            c4 = s * 4
            for q in range(4):
                bcol = jnp.full((16,), 0, jnp.int32) + (c4 + q)
                oref[s, pl.ds(q * 32, 16)] = plsc.load_gather(
                    sref, [elane0 + bcol])
                oref[s, pl.ds(q * 32 + 16, 16)] = plsc.load_gather(
                    sref, [elane16 + bcol])

    def transpose(p, nsr):
        transpose_ref(src[p], obuf[p], nsr, 4 * TSR)

    def fire_out(t, p):
        pltpu.async_copy(
            obuf[p], out_sr.at[pl.ds(t * TSR, TSR)], osem[p])

    def drain_out(p):
        pltpu.make_async_copy(
            obuf[p], out_sr.at[pl.ds(0, TSR)], osem[p]).wait()

    def maybe(t, fn):
        pl.when(t < NT_FULL)(fn)

    maybe(wid, lambda: fire_in(wid, 0))
    maybe(wid + NW, lambda: fire_in(wid + NW, 1))

    def k_body(k, carry):
        for p in (0, 1):
            t = wid + NW * (2 * k + p)
            t_next = t + 2 * NW

            def task(t=t, t_next=t_next, p=p):
                drain_in(p)
                pl.when(k >= 1)(lambda: drain_out(p))
                transpose(p, TSR)
                fire_out(t, p)
                pl.when(t_next < NT_FULL)(lambda: fire_in(t_next, p))
            maybe(t, task)
        return carry

    nk = (NT_FULL + 2 * NW - 1) // (2 * NW)
    lax.fori_loop(0, nk, k_body, 0)
    drain_out(0)
    drain_out(1)

    def tail():
        pltpu.sync_copy(
            tab_t.at[:, pl.ds(4 * NT_FULL * TSR, 4 * TAIL)], tsrc)

        @plsc.parallel_loop(0, TAIL)
        def _(s):
            c4 = s * 4
            for q in range(4):
                col = jnp.full((16,), 0, jnp.int32) + (c4 + q)
                for e0 in (0, 16):
                    vals = plsc.load_gather(tsrc, [e0 + lane, col])
                    ob0[s, pl.ds(q * 32 + e0, 16)] = vals
        pltpu.sync_copy(ob0.at[pl.ds(0, TAIL)],
                        out_sr.at[pl.ds(NT_FULL * TSR, TAIL)])
    pl.when(wid == 0)(tail)


@jax.jit
def _sc_transpose(tab_t):
    mesh = plsc.VectorSubcoreMesh(
        core_axis_name="c", subcore_axis_name="s",
        num_cores=NC, num_subcores=NS)
    return pl.kernel(
        _transpose_body,
        out_type=jax.ShapeDtypeStruct((NSR, 128), jnp.float32),
        mesh=mesh,
        scratch_types=[
            pltpu.VMEM((EMBED_DIM * 4 * TSR,), jnp.float32),
            pltpu.VMEM((EMBED_DIM * 4 * TSR,), jnp.float32),
            pltpu.VMEM((TSR, 128), jnp.float32),
            pltpu.VMEM((TSR, 128), jnp.float32),
            pltpu.VMEM((EMBED_DIM, 4 * TAIL), jnp.float32),
            pltpu.SemaphoreType.DMA,
            pltpu.SemaphoreType.DMA,
            pltpu.SemaphoreType.DMA,
            pltpu.SemaphoreType.DMA,
        ],
        compiler_params=pltpu.CompilerParams(
            use_tc_tiling_on_sc=True, needs_layout_passes=False),
    )(tab_t)


@jax.jit
def _sc_gather(idx_t, table_sr):
    mesh = plsc.VectorSubcoreMesh(
        core_axis_name="c", subcore_axis_name="s",
        num_cores=NC, num_subcores=NS)
    return pl.kernel(
        _gather_body,
        out_type=jax.ShapeDtypeStruct((FIELDS, EMBED_DIM, BATCH), jnp.float32),
        mesh=mesh,
        scratch_types=[
            pltpu.VMEM((KROWS,), jnp.int32),
            pltpu.VMEM((KROWS,), jnp.int32),
            pltpu.VMEM((KROWS,), jnp.int32),
            pltpu.VMEM((KROWS,), jnp.int32),
            pltpu.VMEM((KROWS, 128), jnp.float32),
            pltpu.VMEM((KROWS, 128), jnp.float32),
            pltpu.VMEM((EMBED_DIM, KROWS), jnp.float32),
            pltpu.VMEM((EMBED_DIM, KROWS), jnp.float32),
            pltpu.SemaphoreType.DMA,
            pltpu.SemaphoreType.DMA,
            pltpu.SemaphoreType.DMA,
            pltpu.SemaphoreType.DMA,
        ],
        compiler_params=pltpu.CompilerParams(
            use_tc_tiling_on_sc=True, needs_layout_passes=False),
    )(idx_t, table_sr)


def kernel(idx, table):
    idx_t = idx.T.astype(jnp.int32)
    table_t = table.T
    table_sr = _sc_transpose(table_t)
    out_t = _sc_gather(idx_t, table_sr)
    return out_t.transpose(2, 0, 1)


# speedup vs baseline: 1.0023x; 1.0023x over previous
import jax
import jax.numpy as jnp
from jax import lax
from jax.experimental import pallas as pl
from jax.experimental.pallas import tpu as pltpu
from jax.experimental.pallas import tpu_sc as plsc

NC = 2
NS = 16
NW = NC * NS

EMBED_DIM = 32
BATCH = 16384
FIELDS = 26
KROWS = 256
NBLK = BATCH // KROWS
BLK_PER_W = NBLK // NW
SUB = 128
NSUB = KROWS // SUB
NGRP = KROWS // 16


def _gather_body(idx_hbm, table_hbm, out_hbm,
                 idxv0, idxv1, srv0, srv1, sst0, sst1, tb0, tb1,
                 gsem0, gsem1, osem0, osem1):
    wid = lax.axis_index("s") * NC + lax.axis_index("c")
    idxv = (idxv0, idxv1)
    srv = (srv0, srv1)
    sst = (sst0, sst1)
    tbuf = (tb0, tb1)
    gsem = (gsem0, gsem1)
    osem = (osem0, osem1)
    b0s = (wid * KROWS, (wid + NW) * KROWS)

    lane = lax.broadcasted_iota(jnp.int32, (16,), 0)

    def load_idx(f, p):
        pltpu.sync_copy(idx_hbm.at[f, pl.ds(b0s[p], KROWS)], idxv[p])

    def compute_srows(p):
        @plsc.parallel_loop(0, NGRP, unroll=4)
        def _(g):
            v = idxv[p][pl.ds(g * 16, 16)]
            srv[p][pl.ds(g * 16, 16)] = lax.shift_right_logical(v, 2)

    def fire_gather(p):
        for j in range(NSUB):
            pltpu.async_copy(
                table_hbm.at[srv[p].at[pl.ds(j * SUB, SUB)]],
                sst[p].at[pl.ds(j * SUB, SUB)],
                gsem[p])

    def drain_gather(p):
        pltpu.make_async_copy(
            table_hbm.at[srv[p]], sst[p], gsem[p]).wait()

    def extract(p):
        @plsc.parallel_loop(0, NGRP, unroll=2)
        def _(g):
            iv = idxv[p][pl.ds(g * 16, 16)]
            row = g * 16 + lane
            qcol = lax.shift_left(lax.bitwise_and(iv, 3), 5)
            for e in range(EMBED_DIM):
                vals = plsc.load_gather(sst[p], [row, qcol + e])
                tbuf[p][e, pl.ds(g * 16, 16)] = vals

    def fire_out(f, p):
        pltpu.async_copy(
            tbuf[p], out_hbm.at[f, :, pl.ds(b0s[p], KROWS)], osem[p])

    def drain_out(p):
        pltpu.make_async_copy(
            tbuf[p], out_hbm.at[0, :, pl.ds(b0s[p], KROWS)], osem[p]).wait()

    for p in (0, 1):
        load_idx(0, p)
        compute_srows(p)
        fire_gather(p)

    def field_body(f, carry):
        for p in (0, 1):
            drain_gather(p)
            pl.when(f >= 1)(lambda: drain_out(p))
            extract(p)
            fire_out(f, p)

            def prefetch(p=p):
                load_idx(f + 1, p)
                compute_srows(p)
                fire_gather(p)
            pl.when(f + 1 < FIELDS)(prefetch)
        return carry

    lax.fori_loop(0, FIELDS, field_body, 0)
    drain_out(0)
    drain_out(1)


NSR = 250000
TSR = 128
NT_FULL = NSR // TSR
TAIL = NSR - NT_FULL * TSR


def _transpose_body(tab_t, out_sr, src0, src1, ob0, ob1, tsrc,
                    isem0, isem1, osem0, osem1):
    wid = lax.axis_index("s") * NC + lax.axis_index("c")
    src = (src0, src1)
    obuf = (ob0, ob1)
    isem = (isem0, isem1)
    osem = (osem0, osem1)
    lane = lax.broadcasted_iota(jnp.int32, (16,), 0)

    def fire_in(t, p):
        for e in range(EMBED_DIM):
            pltpu.async_copy(
                tab_t.at[e, pl.ds(t * (4 * TSR), 4 * TSR)],
                src[p].at[pl.ds(e * (4 * TSR), 4 * TSR)], isem[p])

    def drain_in(p):
        pltpu.make_async_copy(
            tab_t.at[0, pl.ds(0, EMBED_DIM * 4 * TSR)], src[p],
            isem[p]).wait()

    def transpose_ref(sref, oref, nsr, stride):
        elane0 = lane * stride
        elane16 = lane * stride + 16 * stride
        @plsc.parallel_loop(0, nsr, unroll=4)
        def _(s):
            c4 = s * 4
            for q in range(4):
                bcol = jnp.full((16,), 0, jnp.int32) + (c4 + q)
                oref[s, pl.ds(q * 32, 16)] = plsc.load_gather(
                    sref, [elane0 + bcol])
                oref[s, pl.ds(q * 32 + 16, 16)] = plsc.load_gather(
                    sref, [elane16 + bcol])

    def transpose(p, nsr):
        transpose_ref(src[p], obuf[p], nsr, 4 * TSR)

    def fire_out(t, p):
        pltpu.async_copy(
            obuf[p], out_sr.at[pl.ds(t * TSR, TSR)], osem[p])

    def drain_out(p):
        pltpu.make_async_copy(
            obuf[p], out_sr.at[pl.ds(0, TSR)], osem[p]).wait()

    def maybe(t, fn):
        pl.when(t < NT_FULL)(fn)

    maybe(wid, lambda: fire_in(wid, 0))
    maybe(wid + NW, lambda: fire_in(wid + NW, 1))

    def k_body(k, carry):
        for p in (0, 1):
            t = wid + NW * (2 * k + p)
            t_next = t + 2 * NW

            def task(t=t, t_next=t_next, p=p):
                drain_in(p)
                pl.when(k >= 1)(lambda: drain_out(p))
                transpose(p, TSR)
                fire_out(t, p)
                pl.when(t_next < NT_FULL)(lambda: fire_in(t_next, p))
            maybe(t, task)
        return carry

    nk = (NT_FULL + 2 * NW - 1) // (2 * NW)
    lax.fori_loop(0, nk, k_body, 0)
    drain_out(0)
    drain_out(1)

    def tail():
        pltpu.sync_copy(
            tab_t.at[:, pl.ds(4 * NT_FULL * TSR, 4 * TAIL)], tsrc)

        @plsc.parallel_loop(0, TAIL)
        def _(s):
            c4 = s * 4
            for q in range(4):
                col = jnp.full((16,), 0, jnp.int32) + (c4 + q)
                for e0 in (0, 16):
                    vals = plsc.load_gather(tsrc, [e0 + lane, col])
                    ob0[s, pl.ds(q * 32 + e0, 16)] = vals
        pltpu.sync_copy(ob0.at[pl.ds(0, TAIL)],
                        out_sr.at[pl.ds(NT_FULL * TSR, TAIL)])
    pl.when(wid == 0)(tail)


@jax.jit
def _sc_transpose(tab_t):
    mesh = plsc.VectorSubcoreMesh(
        core_axis_name="c", subcore_axis_name="s",
        num_cores=NC, num_subcores=NS)
    return pl.kernel(
        _transpose_body,
        out_type=jax.ShapeDtypeStruct((NSR, 128), jnp.float32),
        mesh=mesh,
        scratch_types=[
            pltpu.VMEM((EMBED_DIM * 4 * TSR,), jnp.float32),
            pltpu.VMEM((EMBED_DIM * 4 * TSR,), jnp.float32),
            pltpu.VMEM((TSR, 128), jnp.float32),
            pltpu.VMEM((TSR, 128), jnp.float32),
            pltpu.VMEM((EMBED_DIM, 4 * TAIL), jnp.float32),
            pltpu.SemaphoreType.DMA,
            pltpu.SemaphoreType.DMA,
            pltpu.SemaphoreType.DMA,
            pltpu.SemaphoreType.DMA,
        ],
        compiler_params=pltpu.CompilerParams(
            use_tc_tiling_on_sc=True, needs_layout_passes=False),
    )(tab_t)


@jax.jit
def _sc_gather(idx_t, table_sr):
    mesh = plsc.VectorSubcoreMesh(
        core_axis_name="c", subcore_axis_name="s",
        num_cores=NC, num_subcores=NS)
    return pl.kernel(
        _gather_body,
        out_type=jax.ShapeDtypeStruct((FIELDS, EMBED_DIM, BATCH), jnp.float32),
        mesh=mesh,
        scratch_types=[
            pltpu.VMEM((KROWS,), jnp.int32),
            pltpu.VMEM((KROWS,), jnp.int32),
            pltpu.VMEM((KROWS,), jnp.int32),
            pltpu.VMEM((KROWS,), jnp.int32),
            pltpu.VMEM((KROWS, 128), jnp.float32),
            pltpu.VMEM((KROWS, 128), jnp.float32),
            pltpu.VMEM((EMBED_DIM, KROWS), jnp.float32),
            pltpu.VMEM((EMBED_DIM, KROWS), jnp.float32),
            pltpu.SemaphoreType.DMA,
            pltpu.SemaphoreType.DMA,
            pltpu.SemaphoreType.DMA,
            pltpu.SemaphoreType.DMA,
        ],
        compiler_params=pltpu.CompilerParams(
            use_tc_tiling_on_sc=True, needs_layout_passes=False),
    )(idx_t, table_sr)


def kernel(idx, table):
    idx_t = idx.T.astype(jnp.int32)
    table_t = table.T
    table_sr = _sc_transpose(table_t)
    out_t = _sc_gather(idx_t, table_sr)
    return out_t.transpose(2, 0, 1)
